# baseline (speedup 1.0000x reference)
import functools

import jax
import jax.numpy as jnp
from jax import lax
from jax.experimental import pallas as pl
from jax.experimental.pallas import tpu as pltpu
from jax.experimental.pallas import tpu_sc as plsc

_NC = 2
_NS = 16
_NW = _NC * _NS
_D = 128
_SPG = 2
_NBUF = 4


@functools.cache
def _make_gather(n_seq, seq_len):
    assert n_seq % (_NW * _SPG) == 0
    n_chunks = n_seq // (_NW * _SPG)
    spw = n_chunks * _SPG
    c_rows = _SPG * seq_len
    assert n_chunks % _NBUF == 0 and n_chunks >= 2 * _NBUF
    mesh = plsc.VectorSubcoreMesh(core_axis_name="c", subcore_axis_name="s")

    @functools.partial(
        pl.kernel,
        out_type=jax.ShapeDtypeStruct((n_seq, seq_len, _D), jnp.float32),
        mesh=mesh,
        scratch_types=[
            pltpu.VMEM((n_chunks, c_rows), jnp.int32),
            pltpu.VMEM((_NBUF, c_rows, _D), jnp.float32),
            pltpu.SemaphoreType.DMA((_NBUF,)),
            pltpu.SemaphoreType.DMA((_NBUF,)),
        ],
    )
    def body(table, idx, out, idx_v, rows, gsem, wsem):
        wid = lax.axis_index("s") * _NC + lax.axis_index("c")
        seq0 = wid * spw
        pltpu.sync_copy(idx.at[wid], idx_v)

        def fire_writes(b, j):
            for q in range(_SPG):
                pltpu.async_copy(
                    rows.at[b, pl.ds(q * seq_len, seq_len)],
                    out.at[seq0 + j * _SPG + q],
                    wsem.at[b],
                )

        def wait_writes(b):
            for q in range(_SPG):
                pltpu.make_async_copy(
                    rows.at[b, pl.ds(q * seq_len, seq_len)],
                    out.at[seq0],
                    wsem.at[b],
                ).wait()

        for b in range(_NBUF - 1):
            pltpu.async_copy(table.at[idx_v.at[b]], rows.at[b], gsem.at[b])

        @pl.loop(0, n_chunks, step=_NBUF)
        def _(j0):
            for b in range(_NBUF):
                j = j0 + b
                bn = (b - 1) % _NBUF
                pltpu.make_async_copy(
                    table.at[idx_v.at[j]], rows.at[b], gsem.at[b]
                ).wait()
                fire_writes(b, j)
                jn = j + _NBUF - 1

                @pl.when(jn < n_chunks)
                def _():
                    @pl.when(j >= 1)
                    def _():
                        wait_writes(bn)

                    pltpu.async_copy(
                        table.at[idx_v.at[jn]], rows.at[bn], gsem.at[bn]
                    )

        for b in range(_NBUF):
            wait_writes(b)

    return body


_NSPLIT = 4


def kernel(x, embedding_table):
    n_seq, seq_len = x.shape
    per = n_seq // _NSPLIT
    gather = _make_gather(per, seq_len)
    parts = []
    for i in range(_NSPLIT):
        xs = lax.slice(x, (i * per, 0), ((i + 1) * per, seq_len))
        idx = xs.reshape(_NW, per // (_NW * _SPG), _SPG * seq_len)
        parts.append(gather(embedding_table, idx))
    return jnp.concatenate(parts, axis=0)


# speedup vs baseline: 1.7886x; 1.7886x over previous
import functools

import jax
import jax.numpy as jnp
from jax import lax
from jax.experimental import pallas as pl
from jax.experimental.pallas import tpu as pltpu
from jax.experimental.pallas import tpu_sc as plsc

_NC = 2
_NS = 16
_NW = _NC * _NS
_D = 128
_SPG = 2
_NBUF = 4


@functools.cache
def _make_gather(n_seq, seq_len):
    assert n_seq % (_NW * _SPG) == 0
    n_chunks = n_seq // (_NW * _SPG)
    spw = n_chunks * _SPG
    c_rows = _SPG * seq_len
    assert n_chunks % _NBUF == 0 and n_chunks >= 2 * _NBUF
    mesh = plsc.VectorSubcoreMesh(core_axis_name="c", subcore_axis_name="s")

    @functools.partial(
        pl.kernel,
        out_type=jax.ShapeDtypeStruct((n_seq, seq_len, _D), jnp.float32),
        mesh=mesh,
        scratch_types=[
            pltpu.VMEM((n_chunks, _D), jnp.int32),
            pltpu.VMEM((_NBUF, c_rows, _D), jnp.float32),
            pltpu.SemaphoreType.DMA((_NBUF,)),
            pltpu.SemaphoreType.DMA((_NBUF,)),
        ],
        compiler_params=pltpu.CompilerParams(
            use_tc_tiling_on_sc=True, needs_layout_passes=True
        ),
    )
    def body(table, idx, out, idx_v, rows, gsem, wsem):
        wid = lax.axis_index("s") * _NC + lax.axis_index("c")
        seq0 = wid * spw
        pltpu.sync_copy(idx.at[wid], idx_v)

        def fire_writes(b, j):
            for q in range(_SPG):
                pltpu.async_copy(
                    rows.at[b, pl.ds(q * seq_len, seq_len)],
                    out.at[seq0 + j * _SPG + q],
                    wsem.at[b],
                )

        def wait_writes(b):
            for q in range(_SPG):
                pltpu.make_async_copy(
                    rows.at[b, pl.ds(q * seq_len, seq_len)],
                    out.at[seq0],
                    wsem.at[b],
                ).wait()

        for b in range(_NBUF - 1):
            pltpu.async_copy(
                table.at[idx_v.at[b, pl.ds(0, c_rows)]], rows.at[b], gsem.at[b]
            )

        @pl.loop(0, n_chunks, step=_NBUF)
        def _(j0):
            for b in range(_NBUF):
                j = j0 + b
                bn = (b - 1) % _NBUF
                pltpu.make_async_copy(
                    table.at[idx_v.at[j, pl.ds(0, c_rows)]], rows.at[b], gsem.at[b]
                ).wait()
                fire_writes(b, j)
                jn = j + _NBUF - 1

                @pl.when(jn < n_chunks)
                def _():
                    @pl.when(j >= 1)
                    def _():
                        wait_writes(bn)

                    pltpu.async_copy(
                        table.at[idx_v.at[jn, pl.ds(0, c_rows)]],
                        rows.at[bn],
                        gsem.at[bn],
                    )

        for b in range(_NBUF):
            wait_writes(b)

    return body


def kernel(x, embedding_table):
    n_seq, seq_len = x.shape
    c_rows = _SPG * seq_len
    n_chunks = n_seq // (_NW * _SPG)
    idx = x.reshape(_NW, n_chunks, c_rows)
    idx = jnp.pad(idx, ((0, 0), (0, 0), (0, _D - c_rows)))
    return _make_gather(n_seq, seq_len)(embedding_table, idx)


# speedup vs baseline: 1.8135x; 1.0139x over previous
import functools

import jax
import jax.numpy as jnp
from jax import lax
from jax.experimental import pallas as pl
from jax.experimental.pallas import tpu as pltpu
from jax.experimental.pallas import tpu_sc as plsc

_NC = 2
_NS = 16
_NW = _NC * _NS
_D = 128
_SPG = 2
_NBUF = 8


@functools.cache
def _make_gather(n_seq, seq_len):
    assert n_seq % (_NW * _SPG) == 0
    n_chunks = n_seq // (_NW * _SPG)
    spw = n_chunks * _SPG
    c_rows = _SPG * seq_len
    assert n_chunks % _NBUF == 0 and n_chunks >= 2 * _NBUF
    mesh = plsc.VectorSubcoreMesh(core_axis_name="c", subcore_axis_name="s")

    @functools.partial(
        pl.kernel,
        out_type=jax.ShapeDtypeStruct((n_seq, seq_len, _D), jnp.float32),
        mesh=mesh,
        scratch_types=[
            pltpu.VMEM((n_chunks, c_rows), jnp.int32),
            pltpu.VMEM((_NBUF, c_rows, _D), jnp.float32),
            pltpu.SemaphoreType.DMA((_NBUF,)),
            pltpu.SemaphoreType.DMA((_NBUF,)),
        ],
    )
    def body(table, idx, out, idx_v, rows, gsem, wsem):
        wid = lax.axis_index("s") * _NC + lax.axis_index("c")
        seq0 = wid * spw
        pltpu.sync_copy(idx.at[wid], idx_v)

        def fire_writes(b, j):
            for q in range(_SPG):
                pltpu.async_copy(
                    rows.at[b, pl.ds(q * seq_len, seq_len)],
                    out.at[seq0 + j * _SPG + q],
                    wsem.at[b],
                )

        def wait_writes(b):
            for q in range(_SPG):
                pltpu.make_async_copy(
                    rows.at[b, pl.ds(q * seq_len, seq_len)],
                    out.at[seq0],
                    wsem.at[b],
                ).wait()

        for b in range(_NBUF - 1):
            pltpu.async_copy(table.at[idx_v.at[b]], rows.at[b], gsem.at[b])

        @pl.loop(0, n_chunks, step=_NBUF)
        def _(j0):
            for b in range(_NBUF):
                j = j0 + b
                bn = (b - 1) % _NBUF
                pltpu.make_async_copy(
                    table.at[idx_v.at[j]], rows.at[b], gsem.at[b]
                ).wait()
                fire_writes(b, j)
                jn = j + _NBUF - 1

                @pl.when(jn < n_chunks)
                def _():
                    @pl.when(j >= 1)
                    def _():
                        wait_writes(bn)

                    pltpu.async_copy(
                        table.at[idx_v.at[jn]], rows.at[bn], gsem.at[bn]
                    )

        for b in range(_NBUF):
            wait_writes(b)

    return body


def kernel(x, embedding_table):
    n_seq, seq_len = x.shape
    idx = x.reshape(_NW, n_seq // (_NW * _SPG), _SPG * seq_len)
    return _make_gather(n_seq, seq_len)(embedding_table, idx)


# speedup vs baseline: 1.8137x; 1.0001x over previous
import functools

import jax
import jax.numpy as jnp
from jax import lax
from jax.experimental import pallas as pl
from jax.experimental.pallas import tpu as pltpu
from jax.experimental.pallas import tpu_sc as plsc

_NC = 2
_NS = 16
_NW = _NC * _NS
_D = 128
_SPG = 2
_NBUF = 8


@functools.cache
def _make_gather(n_seq, seq_len):
    assert n_seq % (_NW * _SPG) == 0
    n_chunks = n_seq // (_NW * _SPG)
    spw = n_chunks * _SPG
    c_rows = _SPG * seq_len
    assert n_chunks % _NBUF == 0 and n_chunks >= 2 * _NBUF
    mesh = plsc.VectorSubcoreMesh(core_axis_name="c", subcore_axis_name="s")

    @functools.partial(
        pl.kernel,
        out_type=jax.ShapeDtypeStruct((n_seq, seq_len, _D), jnp.float32),
        mesh=mesh,
        scratch_types=[
            pltpu.VMEM((n_chunks, c_rows), jnp.int32),
            pltpu.VMEM((_NBUF, c_rows, _D), jnp.float32),
            pltpu.SemaphoreType.DMA((_NBUF,)),
            pltpu.SemaphoreType.DMA((_NBUF,)),
        ],
    )
    def body(table, idx, out, idx_v, rows, gsem, wsem):
        wid = lax.axis_index("s") * _NC + lax.axis_index("c")
        seq0 = wid * spw
        pltpu.sync_copy(idx.at[wid], idx_v)

        def fire_writes(b, j):
            for q in range(_SPG):
                pltpu.async_copy(
                    rows.at[b, pl.ds(q * seq_len, seq_len)],
                    out.at[seq0 + j * _SPG + q],
                    wsem.at[b],
                )

        def wait_writes(b):
            for q in range(_SPG):
                pltpu.make_async_copy(
                    rows.at[b, pl.ds(q * seq_len, seq_len)],
                    out.at[seq0],
                    wsem.at[b],
                ).wait()

        for b in range(_NBUF - 1):
            pltpu.async_copy(table.at[idx_v.at[b]], rows.at[b], gsem.at[b])

        @pl.loop(0, n_chunks, step=_NBUF)
        def _(j0):
            for b in range(_NBUF):
                j = j0 + b
                bn = (b - 1) % _NBUF
                pltpu.make_async_copy(
                    table.at[idx_v.at[j]], rows.at[b], gsem.at[b]
                ).wait()
                fire_writes(b, j)
                jn = j + _NBUF - 1

                @pl.when(jn < n_chunks)
                def _():
                    @pl.when(j >= 1)
                    def _():
                        wait_writes(bn)

                    pltpu.async_copy(
                        table.at[idx_v.at[jn]], rows.at[bn], gsem.at[bn]
                    )

        for b in range(_NBUF):
            wait_writes(b)

    return body


def kernel(x, embedding_table):
    n_seq, seq_len = x.shape
    idx = x.astype(jnp.int32).reshape(_NW, n_seq // (_NW * _SPG), _SPG * seq_len)
    return _make_gather(n_seq, seq_len)(embedding_table, idx)
